# Initial kernel scaffold; baseline (speedup 1.0000x reference)
import jax
import jax.numpy as jnp
from jax.experimental import pallas as pl


def kernel(x_primal, x_dual, edge_index_pp, edge_index_dd, edge_index_pd, enc_p_W, enc_p_b, enc_d_W, enc_d_b, Wl, bl, Wr, dec_p_W, dec_p_b, dec_d_W, dec_d_b):
    raise NotImplementedError("write your pallas kernel here")


# speedup vs baseline: 8.6464x; 8.6464x over previous
import functools

import jax
import jax.numpy as jnp
from jax import lax
from jax.experimental import pallas as pl
from jax.experimental.pallas import tpu as pltpu
from jax.experimental.pallas import tpu_sc as plsc

N_NODES = 10000
H = 64
E = 320000

_NC = 2
_NS = 16
_C = 128
_EROWS = 2560
_RPT = _EROWS // _NS
_NACC = 10240
_ZROWS = _NACC // _NS
_OCH = 632
_OLAST = N_NODES - 15 * _OCH
_CW = 8


def _mesh():
    return plsc.VectorSubcoreMesh(core_axis_name="c", subcore_axis_name="s",
                                  num_cores=_NC, num_subcores=_NS)


def _segsum_body(hp, hd, spp, dpp, sdd, ddd, spd, dpd, sdp, ddp, zeros64,
                 o_pp, o_dd, o_pd, o_dp, sidx, didx, rows, acc, sem):
    c = lax.axis_index("c")
    s = lax.axis_index("s")
    rels = (
        (0, spp, dpp, hp, o_pp),
        (0, sdp, ddp, hd, o_dp),
        (1, sdd, ddd, hd, o_dd),
        (1, spd, dpd, hp, o_pd),
    )
    zoff = pl.multiple_of(s * _ZROWS, 8)
    eoff = pl.multiple_of(s * _RPT, 8)
    ooff = pl.multiple_of(s * _OCH, 8)
    for rc, src2d, dst2d, table, out in rels:
        @pl.when(c == rc)
        def _zero_and_stage(src2d=src2d, dst2d=dst2d):
            pltpu.sync_copy(zeros64.at[pl.ds(zoff, _ZROWS)],
                            acc.at[pl.ds(zoff, _ZROWS)])
            pltpu.sync_copy(src2d.at[pl.ds(eoff, _RPT)], sidx)
            pltpu.sync_copy(dst2d.at[pl.ds(eoff, _RPT)], didx)

        plsc.subcore_barrier()

        @pl.when(c == rc)
        def _edges(table=table):
            def body(k, carry):
                pltpu.async_copy(table.at[sidx.at[k]], rows, sem).wait()
                pltpu.sync_copy(rows, acc.at[didx.at[k]], add=True)
                return carry
            lax.fori_loop(0, _RPT, body, 0)

        plsc.subcore_barrier()

        @pl.when((c == rc) & (s < 15))
        def _copy_out(out=out):
            pltpu.sync_copy(acc.at[pl.ds(ooff, _OCH)],
                            out.at[pl.ds(ooff, _OCH)])

        @pl.when((c == rc) & (s == 15))
        def _copy_out_last(out=out):
            pltpu.sync_copy(acc.at[pl.ds(15 * _OCH, _OLAST)],
                            out.at[pl.ds(15 * _OCH, _OLAST)])

        plsc.subcore_barrier()


def _counts_body(dpp, ddd, dpd, ddp, zeros8, ones8,
                 o_pp, o_dd, o_pd, o_dp, didx, onesv, acc, sem):
    del sem
    c = lax.axis_index("c")
    s = lax.axis_index("s")
    pltpu.sync_copy(ones8, onesv)
    rels = ((0, dpp, o_pp), (0, ddp, o_dp), (1, ddd, o_dd), (1, dpd, o_pd))
    zoff = pl.multiple_of(s * _ZROWS, 8)
    eoff = pl.multiple_of(s * _RPT, 8)
    ooff = pl.multiple_of(s * _OCH, 8)
    for rc, dst2d, out in rels:
        @pl.when(c == rc)
        def _zero_and_stage(dst2d=dst2d):
            pltpu.sync_copy(zeros8.at[pl.ds(zoff, _ZROWS)],
                            acc.at[pl.ds(zoff, _ZROWS)])
            pltpu.sync_copy(dst2d.at[pl.ds(eoff, _RPT)], didx)

        plsc.subcore_barrier()

        @pl.when(c == rc)
        def _edges():
            def body(k, carry):
                pltpu.sync_copy(onesv, acc.at[didx.at[k]], add=True)
                return carry
            lax.fori_loop(0, _RPT, body, 0)

        plsc.subcore_barrier()

        @pl.when((c == rc) & (s < 15))
        def _copy_out(out=out):
            pltpu.sync_copy(acc.at[pl.ds(ooff, _OCH)],
                            out.at[pl.ds(ooff, _OCH)])

        @pl.when((c == rc) & (s == 15))
        def _copy_out_last(out=out):
            pltpu.sync_copy(acc.at[pl.ds(15 * _OCH, _OLAST)],
                            out.at[pl.ds(15 * _OCH, _OLAST)])

        plsc.subcore_barrier()


@jax.jit
def _sc_segsum(hp, hd, spp, dpp, sdd, ddd, spd, dpd, sdp, ddp, zeros64):
    f = pl.kernel(
        _segsum_body,
        out_type=[jax.ShapeDtypeStruct((N_NODES, H), jnp.float32)] * 4,
        mesh=_mesh(),
        scratch_types=[
            pltpu.VMEM((_RPT, _C), jnp.int32),
            pltpu.VMEM((_RPT, _C), jnp.int32),
            pltpu.VMEM((_C, H), jnp.float32),
            pltpu.VMEM_SHARED((_NACC, H), jnp.float32),
            pltpu.SemaphoreType.DMA,
        ],
        compiler_params=pltpu.CompilerParams(use_tc_tiling_on_sc=False),
    )
    return f(hp, hd, spp, dpp, sdd, ddd, spd, dpd, sdp, ddp, zeros64)


@jax.jit
def _sc_counts(dpp, ddd, dpd, ddp, zeros8, ones8):
    f = pl.kernel(
        _counts_body,
        out_type=[jax.ShapeDtypeStruct((N_NODES, _CW), jnp.float32)] * 4,
        mesh=_mesh(),
        scratch_types=[
            pltpu.VMEM((_RPT, _C), jnp.int32),
            pltpu.VMEM((_C, _CW), jnp.float32),
            pltpu.VMEM_SHARED((_NACC, _CW), jnp.float32),
            pltpu.SemaphoreType.DMA,
        ],
        compiler_params=pltpu.CompilerParams(use_tc_tiling_on_sc=False),
    )
    return f(dpp, ddd, dpd, ddp, zeros8, ones8)


def _dense(x, W, b, relu):
    n, k = x.shape
    m = W.shape[1]
    bn = 1000

    def body(x_ref, w_ref, b_ref, o_ref):
        acc = jnp.dot(x_ref[...], w_ref[...],
                      preferred_element_type=jnp.float32) + b_ref[...]
        o_ref[...] = jnp.maximum(acc, 0.0) if relu else acc

    return pl.pallas_call(
        body,
        grid=(n // bn,),
        in_specs=[
            pl.BlockSpec((bn, k), lambda i: (i, 0)),
            pl.BlockSpec((k, m), lambda i: (0, 0)),
            pl.BlockSpec((1, m), lambda i: (0, 0)),
        ],
        out_specs=pl.BlockSpec((bn, m), lambda i: (i, 0)),
        out_shape=jax.ShapeDtypeStruct((n, m), jnp.float32),
    )(x, W, b.reshape(1, m))


def _combine(sa, sb, h, ca, cb, wla, wlb, wr2, b2):
    n = h.shape[0]
    bn = 1000

    def body(sa_ref, sb_ref, h_ref, ca_ref, cb_ref, wla_ref, wlb_ref,
             wr_ref, b_ref, o_ref):
        ra = 1.0 / jnp.maximum(ca_ref[...][:, 0:1], 1.0)
        rb = 1.0 / jnp.maximum(cb_ref[...][:, 0:1], 1.0)
        t = jnp.dot(sa_ref[...] * ra, wla_ref[...],
                    preferred_element_type=jnp.float32)
        t = t + jnp.dot(sb_ref[...] * rb, wlb_ref[...],
                        preferred_element_type=jnp.float32)
        t = t + jnp.dot(h_ref[...], wr_ref[...],
                        preferred_element_type=jnp.float32)
        o_ref[...] = jnp.maximum(t + b_ref[...], 0.0)

    return pl.pallas_call(
        body,
        grid=(n // bn,),
        in_specs=[
            pl.BlockSpec((bn, H), lambda i: (i, 0)),
            pl.BlockSpec((bn, H), lambda i: (i, 0)),
            pl.BlockSpec((bn, H), lambda i: (i, 0)),
            pl.BlockSpec((bn, _CW), lambda i: (i, 0)),
            pl.BlockSpec((bn, _CW), lambda i: (i, 0)),
            pl.BlockSpec((H, H), lambda i: (0, 0)),
            pl.BlockSpec((H, H), lambda i: (0, 0)),
            pl.BlockSpec((H, H), lambda i: (0, 0)),
            pl.BlockSpec((1, H), lambda i: (0, 0)),
        ],
        out_specs=pl.BlockSpec((bn, H), lambda i: (i, 0)),
        out_shape=jax.ShapeDtypeStruct((n, H), jnp.float32),
    )(sa, sb, h, ca, cb, wla, wlb, wr2, b2.reshape(1, H))


def _pad_edges(idx, is_dst):
    idx = idx.astype(jnp.int32)
    npad = _EROWS * _C - E
    if is_dst:
        tail = N_NODES + (jnp.arange(npad, dtype=jnp.int32) % (_NACC - N_NODES))
    else:
        tail = jnp.arange(npad, dtype=jnp.int32) % N_NODES
    return jnp.concatenate([idx, tail]).reshape(_EROWS, _C)


def kernel(x_primal, x_dual, edge_index_pp, edge_index_dd, edge_index_pd,
           enc_p_W, enc_p_b, enc_d_W, enc_d_b, Wl, bl, Wr,
           dec_p_W, dec_p_b, dec_d_W, dec_d_b):
    spp = _pad_edges(edge_index_pp[0], False)
    dpp = _pad_edges(edge_index_pp[1], True)
    sdd = _pad_edges(edge_index_dd[0], False)
    ddd = _pad_edges(edge_index_dd[1], True)
    spd = _pad_edges(edge_index_pd[0], False)
    dpd = _pad_edges(edge_index_pd[1], True)
    sdp = _pad_edges(edge_index_pd[1], False)
    ddp = _pad_edges(edge_index_pd[0], True)

    zeros64 = jnp.zeros((_NACC, H), jnp.float32)
    zeros8 = jnp.zeros((_NACC, _CW), jnp.float32)
    ones8 = jnp.ones((_C, _CW), jnp.float32)

    hp = _dense(x_primal, enc_p_W, enc_p_b, relu=True)
    hd = _dense(x_dual, enc_d_W, enc_d_b, relu=True)

    c_pp, c_dd, c_pd, c_dp = _sc_counts(dpp, ddd, dpd, ddp, zeros8, ones8)

    for l in range(2):
        s_pp, s_dd, s_pd, s_dp = _sc_segsum(
            hp, hd, spp, dpp, sdd, ddd, spd, dpd, sdp, ddp, zeros64)
        hp_new = _combine(s_pp, s_dp, hp, c_pp, c_dp,
                          Wl[l, 0], Wl[l, 3], Wr[l, 0] + Wr[l, 3],
                          bl[l, 0] + bl[l, 3])
        hd_new = _combine(s_dd, s_pd, hd, c_dd, c_pd,
                          Wl[l, 1], Wl[l, 2], Wr[l, 1] + Wr[l, 2],
                          bl[l, 1] + bl[l, 2])
        hp, hd = hp_new, hd_new

    out_primal = _dense(hp, dec_p_W, dec_p_b, relu=False)
    out_dual = _dense(hd, dec_d_W, dec_d_b, relu=False)
    return (out_primal, out_dual)


# speedup vs baseline: 14.5683x; 1.6849x over previous
import functools

import jax
import jax.numpy as jnp
from jax import lax
from jax.experimental import pallas as pl
from jax.experimental.pallas import tpu as pltpu
from jax.experimental.pallas import tpu_sc as plsc

N_NODES = 10000
H = 64
E = 320000

_NC = 2
_NS = 16
_C = 128
_EROWS = 2560
_RPT = _EROWS // _NS
_NACC = 10240
_ZROWS = _NACC // _NS
_OCH = 632
_OLAST = N_NODES - 15 * _OCH
_CW = 8
_NB = 8
_ICH = 80
_G = _ICH // _NB
_NCH = _RPT // _ICH


def _mesh():
    return plsc.VectorSubcoreMesh(core_axis_name="c", subcore_axis_name="s",
                                  num_cores=_NC, num_subcores=_NS)


def _segsum_body(hp, hd, spp, dpp, sdd, ddd, spd, dpd, sdp, ddp, zeros64,
                 o_pp, o_dd, o_pd, o_dp, sidx, didx, rows, acc, gsem, ssem):
    c = lax.axis_index("c")
    s = lax.axis_index("s")
    rels = (
        (0, spp, dpp, hp, o_pp),
        (0, sdp, ddp, hd, o_dp),
        (1, sdd, ddd, hd, o_dd),
        (1, spd, dpd, hp, o_pd),
    )
    zoff = pl.multiple_of(s * _ZROWS, 8)
    eoff = pl.multiple_of(s * _RPT, 8)
    ooff = pl.multiple_of(s * _OCH, 8)
    for rc, src2d, dst2d, table, out in rels:
        @pl.when(c == rc)
        def _zero(_=None):
            pltpu.sync_copy(zeros64.at[pl.ds(zoff, _ZROWS)],
                            acc.at[pl.ds(zoff, _ZROWS)])

        plsc.subcore_barrier()

        @pl.when(c == rc)
        def _edges(table=table, src2d=src2d, dst2d=dst2d):
            def chunk(ci, carry):
                coff = pl.multiple_of(eoff + ci * _ICH, 8)
                pltpu.sync_copy(src2d.at[pl.ds(coff, _ICH)], sidx)
                pltpu.sync_copy(dst2d.at[pl.ds(coff, _ICH)], didx)
                for b in range(_NB):
                    pltpu.async_copy(table.at[sidx.at[b]], rows.at[b],
                                     gsem.at[b])

                def outer(g, carry2):
                    for b in range(_NB):
                        k = g * _NB + b
                        pltpu.make_async_copy(table.at[sidx.at[k]],
                                              rows.at[b], gsem.at[b]).wait()
                        pltpu.async_copy(rows.at[b], acc.at[didx.at[k]],
                                         ssem.at[b], add=True)
                    for b in range(_NB):
                        k = g * _NB + b
                        pltpu.make_async_copy(rows.at[b], acc.at[didx.at[k]],
                                              ssem.at[b]).wait()

                        @pl.when(g + 1 < _G)
                        def _next_gather(b=b, g=g):
                            kn = (g + 1) * _NB + b
                            pltpu.async_copy(table.at[sidx.at[kn]],
                                             rows.at[b], gsem.at[b])
                    return carry2

                lax.fori_loop(0, _G, outer, 0)
                return carry

            lax.fori_loop(0, _NCH, chunk, 0)

        plsc.subcore_barrier()

        @pl.when((c == rc) & (s < 15))
        def _copy_out(out=out):
            pltpu.sync_copy(acc.at[pl.ds(ooff, _OCH)],
                            out.at[pl.ds(ooff, _OCH)])

        @pl.when((c == rc) & (s == 15))
        def _copy_out_last(out=out):
            pltpu.sync_copy(acc.at[pl.ds(15 * _OCH, _OLAST)],
                            out.at[pl.ds(15 * _OCH, _OLAST)])

        plsc.subcore_barrier()


def _counts_body(dpp, ddd, dpd, ddp, zeros8, ones8,
                 o_pp, o_dd, o_pd, o_dp, didx, onesv, acc, sem):
    del sem
    c = lax.axis_index("c")
    s = lax.axis_index("s")
    pltpu.sync_copy(ones8, onesv)
    rels = ((0, dpp, o_pp), (0, ddp, o_dp), (1, ddd, o_dd), (1, dpd, o_pd))
    zoff = pl.multiple_of(s * _ZROWS, 8)
    eoff = pl.multiple_of(s * _RPT, 8)
    ooff = pl.multiple_of(s * _OCH, 8)
    for rc, dst2d, out in rels:
        @pl.when(c == rc)
        def _zero_and_stage(dst2d=dst2d):
            pltpu.sync_copy(zeros8.at[pl.ds(zoff, _ZROWS)],
                            acc.at[pl.ds(zoff, _ZROWS)])
            pltpu.sync_copy(dst2d.at[pl.ds(eoff, _RPT)], didx)

        plsc.subcore_barrier()

        @pl.when(c == rc)
        def _edges():
            def body(k, carry):
                pltpu.sync_copy(onesv, acc.at[didx.at[k]], add=True)
                return carry
            lax.fori_loop(0, _RPT, body, 0)

        plsc.subcore_barrier()

        @pl.when((c == rc) & (s < 15))
        def _copy_out(out=out):
            pltpu.sync_copy(acc.at[pl.ds(ooff, _OCH)],
                            out.at[pl.ds(ooff, _OCH)])

        @pl.when((c == rc) & (s == 15))
        def _copy_out_last(out=out):
            pltpu.sync_copy(acc.at[pl.ds(15 * _OCH, _OLAST)],
                            out.at[pl.ds(15 * _OCH, _OLAST)])

        plsc.subcore_barrier()


@jax.jit
def _sc_segsum(hp, hd, spp, dpp, sdd, ddd, spd, dpd, sdp, ddp, zeros64):
    f = pl.kernel(
        _segsum_body,
        out_type=[jax.ShapeDtypeStruct((N_NODES, H), jnp.float32)] * 4,
        mesh=_mesh(),
        scratch_types=[
            pltpu.VMEM((_ICH, _C), jnp.int32),
            pltpu.VMEM((_ICH, _C), jnp.int32),
            pltpu.VMEM((_NB, _C, H), jnp.float32),
            pltpu.VMEM_SHARED((_NACC, H), jnp.float32),
            pltpu.SemaphoreType.DMA((_NB,)),
            pltpu.SemaphoreType.DMA((_NB,)),
        ],
        compiler_params=pltpu.CompilerParams(use_tc_tiling_on_sc=False),
    )
    return f(hp, hd, spp, dpp, sdd, ddd, spd, dpd, sdp, ddp, zeros64)


@jax.jit
def _sc_counts(dpp, ddd, dpd, ddp, zeros8, ones8):
    f = pl.kernel(
        _counts_body,
        out_type=[jax.ShapeDtypeStruct((N_NODES, _CW), jnp.float32)] * 4,
        mesh=_mesh(),
        scratch_types=[
            pltpu.VMEM((_RPT, _C), jnp.int32),
            pltpu.VMEM((_C, _CW), jnp.float32),
            pltpu.VMEM_SHARED((_NACC, _CW), jnp.float32),
            pltpu.SemaphoreType.DMA,
        ],
        compiler_params=pltpu.CompilerParams(use_tc_tiling_on_sc=False),
    )
    return f(dpp, ddd, dpd, ddp, zeros8, ones8)


def _dense(x, W, b, relu):
    n, k = x.shape
    m = W.shape[1]
    bn = 1000

    def body(x_ref, w_ref, b_ref, o_ref):
        acc = jnp.dot(x_ref[...], w_ref[...],
                      preferred_element_type=jnp.float32) + b_ref[...]
        o_ref[...] = jnp.maximum(acc, 0.0) if relu else acc

    return pl.pallas_call(
        body,
        grid=(n // bn,),
        in_specs=[
            pl.BlockSpec((bn, k), lambda i: (i, 0)),
            pl.BlockSpec((k, m), lambda i: (0, 0)),
            pl.BlockSpec((1, m), lambda i: (0, 0)),
        ],
        out_specs=pl.BlockSpec((bn, m), lambda i: (i, 0)),
        out_shape=jax.ShapeDtypeStruct((n, m), jnp.float32),
    )(x, W, b.reshape(1, m))


def _combine(sa, sb, h, ca, cb, wla, wlb, wr2, b2):
    n = h.shape[0]
    bn = 1000

    def body(sa_ref, sb_ref, h_ref, ca_ref, cb_ref, wla_ref, wlb_ref,
             wr_ref, b_ref, o_ref):
        ra = 1.0 / jnp.maximum(ca_ref[...][:, 0:1], 1.0)
        rb = 1.0 / jnp.maximum(cb_ref[...][:, 0:1], 1.0)
        t = jnp.dot(sa_ref[...] * ra, wla_ref[...],
                    preferred_element_type=jnp.float32)
        t = t + jnp.dot(sb_ref[...] * rb, wlb_ref[...],
                        preferred_element_type=jnp.float32)
        t = t + jnp.dot(h_ref[...], wr_ref[...],
                        preferred_element_type=jnp.float32)
        o_ref[...] = jnp.maximum(t + b_ref[...], 0.0)

    return pl.pallas_call(
        body,
        grid=(n // bn,),
        in_specs=[
            pl.BlockSpec((bn, H), lambda i: (i, 0)),
            pl.BlockSpec((bn, H), lambda i: (i, 0)),
            pl.BlockSpec((bn, H), lambda i: (i, 0)),
            pl.BlockSpec((bn, _CW), lambda i: (i, 0)),
            pl.BlockSpec((bn, _CW), lambda i: (i, 0)),
            pl.BlockSpec((H, H), lambda i: (0, 0)),
            pl.BlockSpec((H, H), lambda i: (0, 0)),
            pl.BlockSpec((H, H), lambda i: (0, 0)),
            pl.BlockSpec((1, H), lambda i: (0, 0)),
        ],
        out_specs=pl.BlockSpec((bn, H), lambda i: (i, 0)),
        out_shape=jax.ShapeDtypeStruct((n, H), jnp.float32),
    )(sa, sb, h, ca, cb, wla, wlb, wr2, b2.reshape(1, H))


def _pad_edges(idx, is_dst):
    idx = idx.astype(jnp.int32)
    npad = _EROWS * _C - E
    if is_dst:
        tail = N_NODES + (jnp.arange(npad, dtype=jnp.int32) % (_NACC - N_NODES))
    else:
        tail = jnp.arange(npad, dtype=jnp.int32) % N_NODES
    return jnp.concatenate([idx, tail]).reshape(_EROWS, _C)


def kernel(x_primal, x_dual, edge_index_pp, edge_index_dd, edge_index_pd,
           enc_p_W, enc_p_b, enc_d_W, enc_d_b, Wl, bl, Wr,
           dec_p_W, dec_p_b, dec_d_W, dec_d_b):
    spp = _pad_edges(edge_index_pp[0], False)
    dpp = _pad_edges(edge_index_pp[1], True)
    sdd = _pad_edges(edge_index_dd[0], False)
    ddd = _pad_edges(edge_index_dd[1], True)
    spd = _pad_edges(edge_index_pd[0], False)
    dpd = _pad_edges(edge_index_pd[1], True)
    sdp = _pad_edges(edge_index_pd[1], False)
    ddp = _pad_edges(edge_index_pd[0], True)

    zeros64 = jnp.zeros((_NACC, H), jnp.float32)
    zeros8 = jnp.zeros((_NACC, _CW), jnp.float32)
    ones8 = jnp.ones((_C, _CW), jnp.float32)

    hp = _dense(x_primal, enc_p_W, enc_p_b, relu=True)
    hd = _dense(x_dual, enc_d_W, enc_d_b, relu=True)

    c_pp, c_dd, c_pd, c_dp = _sc_counts(dpp, ddd, dpd, ddp, zeros8, ones8)

    for l in range(2):
        s_pp, s_dd, s_pd, s_dp = _sc_segsum(
            hp, hd, spp, dpp, sdd, ddd, spd, dpd, sdp, ddp, zeros64)
        hp_new = _combine(s_pp, s_dp, hp, c_pp, c_dp,
                          Wl[l, 0], Wl[l, 3], Wr[l, 0] + Wr[l, 3],
                          bl[l, 0] + bl[l, 3])
        hd_new = _combine(s_dd, s_pd, hd, c_dd, c_pd,
                          Wl[l, 1], Wl[l, 2], Wr[l, 1] + Wr[l, 2],
                          bl[l, 1] + bl[l, 2])
        hp, hd = hp_new, hd_new

    out_primal = _dense(hp, dec_p_W, dec_p_b, relu=False)
    out_dual = _dense(hd, dec_d_W, dec_d_b, relu=False)
    return (out_primal, out_dual)


# speedup vs baseline: 14.7357x; 1.0115x over previous
import functools

import jax
import jax.numpy as jnp
from jax import lax
from jax.experimental import pallas as pl
from jax.experimental.pallas import tpu as pltpu
from jax.experimental.pallas import tpu_sc as plsc

N_NODES = 10000
H = 64
E = 320000
OUT_DIM = 128
IN_DIM = 128

_NC = 2
_NS = 16
_C = 128
_EROWS = 2560
_RPT = _EROWS // _NS
_NACC = 10240
_ZROWS = _NACC // _NS
_OCH = 632
_OLAST = N_NODES - 15 * _OCH
_CW = 8
_NB = 8
_ICH = 40
_G = _ICH // _NB
_NCH = _RPT // _ICH


def _mesh():
    return plsc.VectorSubcoreMesh(core_axis_name="c", subcore_axis_name="s",
                                  num_cores=_NC, num_subcores=_NS)


def _segsum_body(with_counts, *refs):
    if with_counts:
        (h, spp, dpp, sdd, ddd, spd, dpd, sdp, ddp, zeros64, zeros8, ones8,
         oA, oB, cA, cB, sidx, didx, rows, onesv, acc, acc8,
         gsem, ssem, csem) = refs
    else:
        (h, spp, dpp, sdd, ddd, spd, dpd, sdp, ddp, zeros64,
         oA, oB, sidx, didx, rows, acc, gsem, ssem) = refs
    c = lax.axis_index("c")
    s = lax.axis_index("s")
    rels = (
        (0, spp, dpp, 0, oA, 0),
        (0, sdp, ddp, 1, oB, 0),
        (1, sdd, ddd, 1, oA, 1),
        (1, spd, dpd, 0, oB, 1),
    )
    zoff = pl.multiple_of(s * _ZROWS, 8)
    eoff = pl.multiple_of(s * _RPT, 8)
    ooff = pl.multiple_of(s * _OCH, 8)

    if with_counts:
        pltpu.sync_copy(ones8, onesv)

    for rc, src2d, dst2d, tslot, out, oslot in rels:
        @pl.when(c == rc)
        def _zero():
            pltpu.sync_copy(zeros64.at[pl.ds(zoff, _ZROWS)],
                            acc.at[pl.ds(zoff, _ZROWS)])
            if with_counts:
                pltpu.sync_copy(zeros8.at[pl.ds(zoff, _ZROWS)],
                                acc8.at[pl.ds(zoff, _ZROWS)])

        plsc.subcore_barrier()

        @pl.when(c == rc)
        def _edges(src2d=src2d, dst2d=dst2d, tslot=tslot):
            table = h.at[tslot]
            def chunk(ci, carry):
                coff = pl.multiple_of(eoff + ci * _ICH, 8)
                pltpu.sync_copy(src2d.at[pl.ds(coff, _ICH)], sidx)
                pltpu.sync_copy(dst2d.at[pl.ds(coff, _ICH)], didx)
                for b in range(_NB):
                    pltpu.async_copy(table.at[sidx.at[b]], rows.at[b],
                                     gsem.at[b])

                def outer(g, carry2):
                    for b in range(_NB):
                        k = g * _NB + b
                        pltpu.make_async_copy(table.at[sidx.at[k]],
                                              rows.at[b], gsem.at[b]).wait()
                        pltpu.async_copy(rows.at[b], acc.at[didx.at[k]],
                                         ssem.at[b], add=True)
                        if with_counts:
                            pltpu.async_copy(onesv, acc8.at[didx.at[k]],
                                             csem.at[b], add=True)
                    for b in range(_NB):
                        k = g * _NB + b
                        pltpu.make_async_copy(rows.at[b], acc.at[didx.at[k]],
                                              ssem.at[b]).wait()
                        if with_counts:
                            pltpu.make_async_copy(
                                onesv, acc8.at[didx.at[k]],
                                csem.at[b]).wait()

                        @pl.when(g + 1 < _G)
                        def _next_gather(b=b, g=g):
                            kn = (g + 1) * _NB + b
                            pltpu.async_copy(table.at[sidx.at[kn]],
                                             rows.at[b], gsem.at[b])
                    return carry2

                lax.fori_loop(0, _G, outer, 0)
                return carry

            lax.fori_loop(0, _NCH, chunk, 0)

        plsc.subcore_barrier()

        @pl.when((c == rc) & (s < 15))
        def _copy_out(out=out, oslot=oslot):
            pltpu.sync_copy(acc.at[pl.ds(ooff, _OCH)],
                            out.at[oslot, pl.ds(ooff, _OCH)])
            if with_counts:
                cout = cA if out is oA else cB
                pltpu.sync_copy(acc8.at[pl.ds(ooff, _OCH)],
                                cout.at[oslot, pl.ds(ooff, _OCH)])

        @pl.when((c == rc) & (s == 15))
        def _copy_out_last(out=out, oslot=oslot):
            pltpu.sync_copy(acc.at[pl.ds(15 * _OCH, _OLAST)],
                            out.at[oslot, pl.ds(15 * _OCH, _OLAST)])
            if with_counts:
                cout = cA if out is oA else cB
                pltpu.sync_copy(acc8.at[pl.ds(15 * _OCH, _OLAST)],
                                cout.at[oslot, pl.ds(15 * _OCH, _OLAST)])

        plsc.subcore_barrier()


@jax.jit
def _sc_segsum0(h, spp, dpp, sdd, ddd, spd, dpd, sdp, ddp,
                zeros64, zeros8, ones8):
    f = pl.kernel(
        functools.partial(_segsum_body, True),
        out_type=[
            jax.ShapeDtypeStruct((2, N_NODES, H), jnp.float32),
            jax.ShapeDtypeStruct((2, N_NODES, H), jnp.float32),
            jax.ShapeDtypeStruct((2, N_NODES, _CW), jnp.float32),
            jax.ShapeDtypeStruct((2, N_NODES, _CW), jnp.float32),
        ],
        mesh=_mesh(),
        scratch_types=[
            pltpu.VMEM((_ICH, _C), jnp.int32),
            pltpu.VMEM((_ICH, _C), jnp.int32),
            pltpu.VMEM((_NB, _C, H), jnp.float32),
            pltpu.VMEM((_C, _CW), jnp.float32),
            pltpu.VMEM_SHARED((_NACC, H), jnp.float32),
            pltpu.VMEM_SHARED((_NACC, _CW), jnp.float32),
            pltpu.SemaphoreType.DMA((_NB,)),
            pltpu.SemaphoreType.DMA((_NB,)),
            pltpu.SemaphoreType.DMA((_NB,)),
        ],
        compiler_params=pltpu.CompilerParams(use_tc_tiling_on_sc=False),
    )
    return f(h, spp, dpp, sdd, ddd, spd, dpd, sdp, ddp, zeros64, zeros8,
             ones8)


@jax.jit
def _sc_segsum1(h, spp, dpp, sdd, ddd, spd, dpd, sdp, ddp, zeros64):
    f = pl.kernel(
        functools.partial(_segsum_body, False),
        out_type=[
            jax.ShapeDtypeStruct((2, N_NODES, H), jnp.float32),
            jax.ShapeDtypeStruct((2, N_NODES, H), jnp.float32),
        ],
        mesh=_mesh(),
        scratch_types=[
            pltpu.VMEM((_ICH, _C), jnp.int32),
            pltpu.VMEM((_ICH, _C), jnp.int32),
            pltpu.VMEM((_NB, _C, H), jnp.float32),
            pltpu.VMEM_SHARED((_NACC, H), jnp.float32),
            pltpu.SemaphoreType.DMA((_NB,)),
            pltpu.SemaphoreType.DMA((_NB,)),
        ],
        compiler_params=pltpu.CompilerParams(use_tc_tiling_on_sc=False),
    )
    return f(h, spp, dpp, sdd, ddd, spd, dpd, sdp, ddp, zeros64)


_BN = 1000


def _enc(x, W, b):
    def body(x_ref, w_ref, b_ref, o_ref):
        acc = jnp.dot(x_ref[0], w_ref[0],
                      preferred_element_type=jnp.float32) + b_ref[0]
        o_ref[0] = jnp.maximum(acc, 0.0)

    return pl.pallas_call(
        body,
        grid=(2, N_NODES // _BN),
        in_specs=[
            pl.BlockSpec((1, _BN, IN_DIM), lambda t, i: (t, i, 0)),
            pl.BlockSpec((1, IN_DIM, H), lambda t, i: (t, 0, 0)),
            pl.BlockSpec((1, 1, H), lambda t, i: (t, 0, 0)),
        ],
        out_specs=pl.BlockSpec((1, _BN, H), lambda t, i: (t, i, 0)),
        out_shape=jax.ShapeDtypeStruct((2, N_NODES, H), jnp.float32),
    )(x, W, b)


def _combine(sA, sB, h, cA, cB, wA, wB, wR, b2, decW=None, decb=None):
    decode = decW is not None
    m = OUT_DIM if decode else H

    def body(*refs):
        if decode:
            (sa, sb, hh, ca, cb, wa, wb, wr, bb, dw, db, o) = refs
        else:
            (sa, sb, hh, ca, cb, wa, wb, wr, bb, o) = refs
        ra = 1.0 / jnp.maximum(ca[0][:, 0:1], 1.0)
        rb = 1.0 / jnp.maximum(cb[0][:, 0:1], 1.0)
        t = jnp.dot(sa[0] * ra, wa[0], preferred_element_type=jnp.float32)
        t = t + jnp.dot(sb[0] * rb, wb[0], preferred_element_type=jnp.float32)
        t = t + jnp.dot(hh[0], wr[0], preferred_element_type=jnp.float32)
        t = jnp.maximum(t + bb[0], 0.0)
        if decode:
            t = jnp.dot(t, dw[0], preferred_element_type=jnp.float32) + db[0]
        o[0] = t

    in_specs = [
        pl.BlockSpec((1, _BN, H), lambda t, i: (t, i, 0)),
        pl.BlockSpec((1, _BN, H), lambda t, i: (t, i, 0)),
        pl.BlockSpec((1, _BN, H), lambda t, i: (t, i, 0)),
        pl.BlockSpec((1, _BN, _CW), lambda t, i: (t, i, 0)),
        pl.BlockSpec((1, _BN, _CW), lambda t, i: (t, i, 0)),
        pl.BlockSpec((1, H, H), lambda t, i: (t, 0, 0)),
        pl.BlockSpec((1, H, H), lambda t, i: (t, 0, 0)),
        pl.BlockSpec((1, H, H), lambda t, i: (t, 0, 0)),
        pl.BlockSpec((1, 1, H), lambda t, i: (t, 0, 0)),
    ]
    args = [sA, sB, h, cA, cB, wA, wB, wR, b2]
    if decode:
        in_specs += [
            pl.BlockSpec((1, H, OUT_DIM), lambda t, i: (t, 0, 0)),
            pl.BlockSpec((1, 1, OUT_DIM), lambda t, i: (t, 0, 0)),
        ]
        args += [decW, decb]

    return pl.pallas_call(
        body,
        grid=(2, N_NODES // _BN),
        in_specs=in_specs,
        out_specs=pl.BlockSpec((1, _BN, m), lambda t, i: (t, i, 0)),
        out_shape=jax.ShapeDtypeStruct((2, N_NODES, m), jnp.float32),
    )(*args)


def _pad_edges(idx, is_dst):
    idx = idx.astype(jnp.int32)
    npad = _EROWS * _C - E
    if is_dst:
        tail = N_NODES + (jnp.arange(npad, dtype=jnp.int32)
                          % (_NACC - N_NODES))
    else:
        tail = jnp.arange(npad, dtype=jnp.int32) % N_NODES
    return jnp.concatenate([idx, tail]).reshape(_EROWS, _C)


def kernel(x_primal, x_dual, edge_index_pp, edge_index_dd, edge_index_pd,
           enc_p_W, enc_p_b, enc_d_W, enc_d_b, Wl, bl, Wr,
           dec_p_W, dec_p_b, dec_d_W, dec_d_b):
    spp = _pad_edges(edge_index_pp[0], False)
    dpp = _pad_edges(edge_index_pp[1], True)
    sdd = _pad_edges(edge_index_dd[0], False)
    ddd = _pad_edges(edge_index_dd[1], True)
    spd = _pad_edges(edge_index_pd[0], False)
    dpd = _pad_edges(edge_index_pd[1], True)
    sdp = _pad_edges(edge_index_pd[1], False)
    ddp = _pad_edges(edge_index_pd[0], True)

    zeros64 = jnp.zeros((_NACC, H), jnp.float32)
    zeros8 = jnp.zeros((_NACC, _CW), jnp.float32)
    ones8 = jnp.ones((_C, _CW), jnp.float32)

    x_st = jnp.stack([x_primal, x_dual])
    encW = jnp.stack([enc_p_W, enc_d_W])
    encb = jnp.stack([enc_p_b, enc_d_b]).reshape(2, 1, H)
    h = _enc(x_st, encW, encb)

    wA = [jnp.stack([Wl[l, 0], Wl[l, 1]]) for l in range(2)]
    wB = [jnp.stack([Wl[l, 3], Wl[l, 2]]) for l in range(2)]
    wR = [jnp.stack([Wr[l, 0] + Wr[l, 3], Wr[l, 1] + Wr[l, 2]])
          for l in range(2)]
    b2 = [jnp.stack([bl[l, 0] + bl[l, 3],
                     bl[l, 1] + bl[l, 2]]).reshape(2, 1, H)
          for l in range(2)]
    decW = jnp.stack([dec_p_W, dec_d_W])
    decb = jnp.stack([dec_p_b, dec_d_b]).reshape(2, 1, OUT_DIM)

    sA, sB, cA, cB = _sc_segsum0(h, spp, dpp, sdd, ddd, spd, dpd, sdp, ddp,
                                 zeros64, zeros8, ones8)
    h = _combine(sA, sB, h, cA, cB, wA[0], wB[0], wR[0], b2[0])
    sA, sB = _sc_segsum1(h, spp, dpp, sdd, ddd, spd, dpd, sdp, ddp, zeros64)
    out = _combine(sA, sB, h, cA, cB, wA[1], wB[1], wR[1], b2[1],
                   decW=decW, decb=decb)
    return (out[0], out[1])


# speedup vs baseline: 15.5999x; 1.0586x over previous
import functools

import jax
import jax.numpy as jnp
from jax import lax
from jax.experimental import pallas as pl
from jax.experimental.pallas import tpu as pltpu
from jax.experimental.pallas import tpu_sc as plsc

N_NODES = 10000
H = 64
E = 320000
OUT_DIM = 128
IN_DIM = 128

_NC = 2
_NS = 16
_C = 128
_EROWS = 2560
_RPT = _EROWS // _NS
_NACC = 10240
_ZROWS = _NACC // _NS
_CW = 8
_NB = 8


def _mesh():
    return plsc.VectorSubcoreMesh(core_axis_name="c", subcore_axis_name="s",
                                  num_cores=_NC, num_subcores=_NS)


def _segsum_body(with_counts, ich, h, epp, edd, epd, *refs):
    if with_counts:
        (zeros64, zeros8, ones8, oA, oB, cA, cB,
         sidx, didx, rows, onesv, acc, acc8, gsem, ssem, csem) = refs
    else:
        (zeros64, oA, oB, sidx, didx, rows, acc, gsem, ssem) = refs
    nch = _RPT // ich
    g_iters = ich // _NB
    c = lax.axis_index("c")
    s = lax.axis_index("s")
    rels = (
        (0, epp, 0, 1, 0, "A", 0),
        (0, epd, 1, 0, 1, "B", 0),
        (1, edd, 0, 1, 1, "A", 1),
        (1, epd, 0, 1, 0, "B", 1),
    )
    zoff = pl.multiple_of(s * _ZROWS, 8)
    eoff = pl.multiple_of(s * _RPT, 8)

    if with_counts:
        pltpu.sync_copy(ones8, onesv)

    for rc, earr, srow, drow, tslot, outn, oslot in rels:
        out = oA if outn == "A" else oB

        @pl.when(c == rc)
        def _zero():
            pltpu.sync_copy(zeros64.at[pl.ds(zoff, _ZROWS)],
                            acc.at[pl.ds(zoff, _ZROWS)])
            if with_counts:
                pltpu.sync_copy(zeros8.at[pl.ds(zoff, _ZROWS)],
                                acc8.at[pl.ds(zoff, _ZROWS)])

        plsc.subcore_barrier()

        @pl.when(c == rc)
        def _edges(earr=earr, srow=srow, drow=drow, tslot=tslot):
            table = h.at[tslot]
            def chunk(ci, carry):
                coff = pl.multiple_of(eoff + ci * ich, 8)
                pltpu.sync_copy(earr.at[srow, pl.ds(coff, ich)], sidx)
                pltpu.sync_copy(earr.at[drow, pl.ds(coff, ich)], didx)
                for b in range(_NB):
                    pltpu.async_copy(table.at[sidx.at[b]], rows.at[b],
                                     gsem.at[b])

                def outer(g, carry2):
                    for b in range(_NB):
                        k = g * _NB + b
                        pltpu.make_async_copy(table.at[sidx.at[k]],
                                              rows.at[b], gsem.at[b]).wait()
                        pltpu.async_copy(rows.at[b], acc.at[didx.at[k]],
                                         ssem.at[b], add=True)
                        if with_counts:
                            pltpu.async_copy(onesv, acc8.at[didx.at[k]],
                                             csem.at[b], add=True)
                    for b in range(_NB):
                        k = g * _NB + b
                        pltpu.make_async_copy(rows.at[b], acc.at[didx.at[k]],
                                              ssem.at[b]).wait()
                        if with_counts:
                            pltpu.make_async_copy(
                                onesv, acc8.at[didx.at[k]],
                                csem.at[b]).wait()

                        @pl.when(g + 1 < g_iters)
                        def _next_gather(b=b, g=g):
                            kn = (g + 1) * _NB + b
                            pltpu.async_copy(table.at[sidx.at[kn]],
                                             rows.at[b], gsem.at[b])
                    return carry2

                lax.fori_loop(0, g_iters, outer, 0)
                return carry

            lax.fori_loop(0, nch, chunk, 0)

        plsc.subcore_barrier()

        @pl.when(c == rc)
        def _copy_out(out=out, oslot=oslot, outn=outn):
            pltpu.sync_copy(acc.at[pl.ds(zoff, _ZROWS)],
                            out.at[oslot, pl.ds(zoff, _ZROWS)])
            if with_counts:
                cout = cA if outn == "A" else cB
                pltpu.sync_copy(acc8.at[pl.ds(zoff, _ZROWS)],
                                cout.at[oslot, pl.ds(zoff, _ZROWS)])

        plsc.subcore_barrier()


@jax.jit
def _sc_segsum0(h, epp, edd, epd, zeros64, zeros8, ones8):
    ich = 40
    f = pl.kernel(
        functools.partial(_segsum_body, True, ich),
        out_type=[
            jax.ShapeDtypeStruct((2, _NACC, H), jnp.float32),
            jax.ShapeDtypeStruct((2, _NACC, H), jnp.float32),
            jax.ShapeDtypeStruct((2, _NACC, _CW), jnp.float32),
            jax.ShapeDtypeStruct((2, _NACC, _CW), jnp.float32),
        ],
        mesh=_mesh(),
        scratch_types=[
            pltpu.VMEM((ich, _C), jnp.int32),
            pltpu.VMEM((ich, _C), jnp.int32),
            pltpu.VMEM((_NB, _C, H), jnp.float32),
            pltpu.VMEM((_C, _CW), jnp.float32),
            pltpu.VMEM_SHARED((_NACC, H), jnp.float32),
            pltpu.VMEM_SHARED((_NACC, _CW), jnp.float32),
            pltpu.SemaphoreType.DMA((_NB,)),
            pltpu.SemaphoreType.DMA((_NB,)),
            pltpu.SemaphoreType.DMA((_NB,)),
        ],
        compiler_params=pltpu.CompilerParams(use_tc_tiling_on_sc=False),
    )
    return f(h, epp, edd, epd, zeros64, zeros8, ones8)


@jax.jit
def _sc_segsum1(h, epp, edd, epd, zeros64):
    ich = 80
    f = pl.kernel(
        functools.partial(_segsum_body, False, ich),
        out_type=[
            jax.ShapeDtypeStruct((2, _NACC, H), jnp.float32),
            jax.ShapeDtypeStruct((2, _NACC, H), jnp.float32),
        ],
        mesh=_mesh(),
        scratch_types=[
            pltpu.VMEM((ich, _C), jnp.int32),
            pltpu.VMEM((ich, _C), jnp.int32),
            pltpu.VMEM((_NB, _C, H), jnp.float32),
            pltpu.VMEM_SHARED((_NACC, H), jnp.float32),
            pltpu.SemaphoreType.DMA((_NB,)),
            pltpu.SemaphoreType.DMA((_NB,)),
        ],
        compiler_params=pltpu.CompilerParams(use_tc_tiling_on_sc=False),
    )
    return f(h, epp, edd, epd, zeros64)


_BN = 1000


def _enc(x, W, b):
    def body(x_ref, w_ref, b_ref, o_ref):
        acc = jnp.dot(x_ref[0], w_ref[0],
                      preferred_element_type=jnp.float32) + b_ref[0]
        o_ref[0] = jnp.maximum(acc, 0.0)

    return pl.pallas_call(
        body,
        grid=(2, N_NODES // _BN),
        in_specs=[
            pl.BlockSpec((1, _BN, IN_DIM), lambda t, i: (t, i, 0)),
            pl.BlockSpec((1, IN_DIM, H), lambda t, i: (t, 0, 0)),
            pl.BlockSpec((1, 1, H), lambda t, i: (t, 0, 0)),
        ],
        out_specs=pl.BlockSpec((1, _BN, H), lambda t, i: (t, i, 0)),
        out_shape=jax.ShapeDtypeStruct((2, _NACC, H), jnp.float32),
    )(x, W, b)


def _combine(sA, sB, h, cA, cB, wA, wB, wR, b2, decW=None, decb=None):
    decode = decW is not None
    m = OUT_DIM if decode else H
    out_rows = N_NODES if decode else _NACC

    def body(*refs):
        if decode:
            (sa, sb, hh, ca, cb, wa, wb, wr, bb, dw, db, o) = refs
        else:
            (sa, sb, hh, ca, cb, wa, wb, wr, bb, o) = refs
        ra = 1.0 / jnp.maximum(ca[0][:, 0:1], 1.0)
        rb = 1.0 / jnp.maximum(cb[0][:, 0:1], 1.0)
        t = jnp.dot(sa[0] * ra, wa[0], preferred_element_type=jnp.float32)
        t = t + jnp.dot(sb[0] * rb, wb[0], preferred_element_type=jnp.float32)
        t = t + jnp.dot(hh[0], wr[0], preferred_element_type=jnp.float32)
        t = jnp.maximum(t + bb[0], 0.0)
        if decode:
            t = jnp.dot(t, dw[0], preferred_element_type=jnp.float32) + db[0]
        o[0] = t

    in_specs = [
        pl.BlockSpec((1, _BN, H), lambda t, i: (t, i, 0)),
        pl.BlockSpec((1, _BN, H), lambda t, i: (t, i, 0)),
        pl.BlockSpec((1, _BN, H), lambda t, i: (t, i, 0)),
        pl.BlockSpec((1, _BN, _CW), lambda t, i: (t, i, 0)),
        pl.BlockSpec((1, _BN, _CW), lambda t, i: (t, i, 0)),
        pl.BlockSpec((1, H, H), lambda t, i: (t, 0, 0)),
        pl.BlockSpec((1, H, H), lambda t, i: (t, 0, 0)),
        pl.BlockSpec((1, H, H), lambda t, i: (t, 0, 0)),
        pl.BlockSpec((1, 1, H), lambda t, i: (t, 0, 0)),
    ]
    args = [sA, sB, h, cA, cB, wA, wB, wR, b2]
    if decode:
        in_specs += [
            pl.BlockSpec((1, H, OUT_DIM), lambda t, i: (t, 0, 0)),
            pl.BlockSpec((1, 1, OUT_DIM), lambda t, i: (t, 0, 0)),
        ]
        args += [decW, decb]

    return pl.pallas_call(
        body,
        grid=(2, N_NODES // _BN),
        in_specs=in_specs,
        out_specs=pl.BlockSpec((1, _BN, m), lambda t, i: (t, i, 0)),
        out_shape=jax.ShapeDtypeStruct((2, out_rows, m), jnp.float32),
    )(*args)


def _pad_edges(ei):
    npad = _EROWS * _C - E
    tail = N_NODES + (jnp.arange(npad, dtype=jnp.int32)
                      % (_NACC - N_NODES))
    tail = jnp.broadcast_to(tail, (2, npad))
    return jnp.concatenate([ei.astype(jnp.int32), tail],
                           axis=1).reshape(2, _EROWS, _C)


def kernel(x_primal, x_dual, edge_index_pp, edge_index_dd, edge_index_pd,
           enc_p_W, enc_p_b, enc_d_W, enc_d_b, Wl, bl, Wr,
           dec_p_W, dec_p_b, dec_d_W, dec_d_b):
    epp = _pad_edges(edge_index_pp)
    edd = _pad_edges(edge_index_dd)
    epd = _pad_edges(edge_index_pd)

    zeros64 = jnp.zeros((_NACC, H), jnp.float32)
    zeros8 = jnp.zeros((_NACC, _CW), jnp.float32)
    ones8 = jnp.ones((_C, _CW), jnp.float32)

    x_st = jnp.stack([x_primal, x_dual])
    encW = jnp.stack([enc_p_W, enc_d_W])
    encb = jnp.stack([enc_p_b, enc_d_b]).reshape(2, 1, H)
    h = _enc(x_st, encW, encb)

    wA = [jnp.stack([Wl[l, 0], Wl[l, 1]]) for l in range(2)]
    wB = [jnp.stack([Wl[l, 3], Wl[l, 2]]) for l in range(2)]
    wR = [jnp.stack([Wr[l, 0] + Wr[l, 3], Wr[l, 1] + Wr[l, 2]])
          for l in range(2)]
    b2 = [jnp.stack([bl[l, 0] + bl[l, 3],
                     bl[l, 1] + bl[l, 2]]).reshape(2, 1, H)
          for l in range(2)]
    decW = jnp.stack([dec_p_W, dec_d_W])
    decb = jnp.stack([dec_p_b, dec_d_b]).reshape(2, 1, OUT_DIM)

    sA, sB, cA, cB = _sc_segsum0(h, epp, edd, epd, zeros64, zeros8, ones8)
    h = _combine(sA, sB, h, cA, cB, wA[0], wB[0], wR[0], b2[0])
    sA, sB = _sc_segsum1(h, epp, edd, epd, zeros64)
    out = _combine(sA, sB, h, cA, cB, wA[1], wB[1], wR[1], b2[1],
                   decW=decW, decb=decb)
    return (out[0], out[1])


# speedup vs baseline: 16.1079x; 1.0326x over previous
import functools

import jax
import jax.numpy as jnp
from jax import lax
from jax.experimental import pallas as pl
from jax.experimental.pallas import tpu as pltpu
from jax.experimental.pallas import tpu_sc as plsc

N_NODES = 10000
H = 64
E = 320000
OUT_DIM = 128
IN_DIM = 128

_NC = 2
_NS = 16
_C = 128
_EROWS = 2560
_RPT = _EROWS // _NS
_NACC = 10240
_ZROWS = _NACC // _NS
_CW = 8
_NB = 8


def _mesh():
    return plsc.VectorSubcoreMesh(core_axis_name="c", subcore_axis_name="s",
                                  num_cores=_NC, num_subcores=_NS)


def _segsum_body(with_counts, ich, h, epp, edd, epd, *refs):
    if with_counts:
        (zeros64, zeros8, ones8, oA, oB, cA, cB,
         sidx, didx, rows, onesv, acc, acc8, gsem, ssem, csem) = refs
    else:
        (zeros64, oA, oB, sidx, didx, rows, acc, gsem, ssem) = refs
    nch = _RPT // ich
    g_iters = ich // _NB
    c = lax.axis_index("c")
    s = lax.axis_index("s")
    rels = (
        (0, epp, 0, 1, 0, "A", 0),
        (0, epd, 1, 0, 1, "B", 0),
        (1, edd, 0, 1, 1, "A", 1),
        (1, epd, 0, 1, 0, "B", 1),
    )
    zoff = pl.multiple_of(s * _ZROWS, 8)
    eoff = pl.multiple_of(s * _RPT, 8)

    if with_counts:
        pltpu.sync_copy(ones8, onesv)

    for rc, earr, srow, drow, tslot, outn, oslot in rels:
        out = oA if outn == "A" else oB

        @pl.when(c == rc)
        def _zero():
            pltpu.sync_copy(zeros64.at[pl.ds(zoff, _ZROWS)],
                            acc.at[pl.ds(zoff, _ZROWS)])
            if with_counts:
                pltpu.sync_copy(zeros8.at[pl.ds(zoff, _ZROWS)],
                                acc8.at[pl.ds(zoff, _ZROWS)])

        plsc.subcore_barrier()

        @pl.when(c == rc)
        def _edges(earr=earr, srow=srow, drow=drow, tslot=tslot):
            table = h.at[tslot]
            def chunk(ci, carry):
                coff = pl.multiple_of(eoff + ci * ich, 8)
                pltpu.sync_copy(earr.at[srow, pl.ds(coff, ich)], sidx)
                pltpu.sync_copy(earr.at[drow, pl.ds(coff, ich)], didx)
                for b in range(_NB):
                    pltpu.async_copy(table.at[sidx.at[b]], rows.at[b],
                                     gsem.at[b])

                def outer(g, carry2):
                    for b in range(_NB):
                        k = g * _NB + b
                        pltpu.make_async_copy(table.at[sidx.at[k]],
                                              rows.at[b], gsem.at[b]).wait()
                        pltpu.async_copy(rows.at[b], acc.at[didx.at[k]],
                                         ssem.at[b], add=True)
                        if with_counts:
                            pltpu.async_copy(onesv, acc8.at[didx.at[k]],
                                             csem.at[b], add=True)
                    for b in range(_NB):
                        k = g * _NB + b
                        pltpu.make_async_copy(rows.at[b], acc.at[didx.at[k]],
                                              ssem.at[b]).wait()
                        if with_counts:
                            pltpu.make_async_copy(
                                onesv, acc8.at[didx.at[k]],
                                csem.at[b]).wait()

                        @pl.when(g + 1 < g_iters)
                        def _next_gather(b=b, g=g):
                            kn = (g + 1) * _NB + b
                            pltpu.async_copy(table.at[sidx.at[kn]],
                                             rows.at[b], gsem.at[b])
                    return carry2

                lax.fori_loop(0, g_iters, outer, 0)
                return carry

            lax.fori_loop(0, nch, chunk, 0)

        plsc.subcore_barrier()

        @pl.when(c == rc)
        def _copy_out(out=out, oslot=oslot, outn=outn):
            pltpu.sync_copy(acc.at[pl.ds(zoff, _ZROWS)],
                            out.at[oslot, pl.ds(zoff, _ZROWS)])
            if with_counts:
                cout = cA if outn == "A" else cB
                pltpu.sync_copy(acc8.at[pl.ds(zoff, _ZROWS)],
                                cout.at[oslot, pl.ds(zoff, _ZROWS)])

        plsc.subcore_barrier()


@jax.jit
def _sc_segsum0(h, epp, edd, epd, zeros64, zeros8, ones8):
    ich = 40
    f = pl.kernel(
        functools.partial(_segsum_body, True, ich),
        out_type=[
            jax.ShapeDtypeStruct((2, _NACC, H), jnp.float32),
            jax.ShapeDtypeStruct((2, _NACC, H), jnp.float32),
            jax.ShapeDtypeStruct((2, _NACC, _CW), jnp.float32),
            jax.ShapeDtypeStruct((2, _NACC, _CW), jnp.float32),
        ],
        mesh=_mesh(),
        scratch_types=[
            pltpu.VMEM((ich, _C), jnp.int32),
            pltpu.VMEM((ich, _C), jnp.int32),
            pltpu.VMEM((_NB, _C, H), jnp.float32),
            pltpu.VMEM((_C, _CW), jnp.float32),
            pltpu.VMEM_SHARED((_NACC, H), jnp.float32),
            pltpu.VMEM_SHARED((_NACC, _CW), jnp.float32),
            pltpu.SemaphoreType.DMA((_NB,)),
            pltpu.SemaphoreType.DMA((_NB,)),
            pltpu.SemaphoreType.DMA((_NB,)),
        ],
        compiler_params=pltpu.CompilerParams(use_tc_tiling_on_sc=False),
    )
    return f(h, epp, edd, epd, zeros64, zeros8, ones8)


@jax.jit
def _sc_segsum1(h, epp, edd, epd, zeros64):
    ich = 80
    f = pl.kernel(
        functools.partial(_segsum_body, False, ich),
        out_type=[
            jax.ShapeDtypeStruct((2, _NACC, H), jnp.float32),
            jax.ShapeDtypeStruct((2, _NACC, H), jnp.float32),
        ],
        mesh=_mesh(),
        scratch_types=[
            pltpu.VMEM((ich, _C), jnp.int32),
            pltpu.VMEM((ich, _C), jnp.int32),
            pltpu.VMEM((_NB, _C, H), jnp.float32),
            pltpu.VMEM_SHARED((_NACC, H), jnp.float32),
            pltpu.SemaphoreType.DMA((_NB,)),
            pltpu.SemaphoreType.DMA((_NB,)),
        ],
        compiler_params=pltpu.CompilerParams(use_tc_tiling_on_sc=False),
    )
    return f(h, epp, edd, epd, zeros64)


_BN = 2000
_BP = _BN // 2
_NPR = _NACC // 2


def _enc(xp, W, b):
    def body(x_ref, w_ref, b_ref, o_ref):
        halves = []
        for lo in (0, IN_DIM):
            t = jnp.dot(x_ref[0][:, lo:lo + IN_DIM], w_ref[0],
                        preferred_element_type=jnp.float32) + b_ref[0]
            halves.append(jnp.maximum(t, 0.0))
        o_ref[0] = jnp.concatenate(halves, axis=1)

    return pl.pallas_call(
        body,
        grid=(2, N_NODES // _BN),
        in_specs=[
            pl.BlockSpec((1, _BP, 2 * IN_DIM), lambda t, i: (t, i, 0)),
            pl.BlockSpec((1, IN_DIM, H), lambda t, i: (t, 0, 0)),
            pl.BlockSpec((1, 1, H), lambda t, i: (t, 0, 0)),
        ],
        out_specs=pl.BlockSpec((1, _BP, 2 * H), lambda t, i: (t, i, 0)),
        out_shape=jax.ShapeDtypeStruct((2, _NPR, 2 * H), jnp.float32),
    )(xp, W, b)


def _combine(layer, sA, sB, h, raA, raB, Wl, bl, Wr, decW=None, decb=None):
    decode = decW is not None
    l = layer

    def body(*refs):
        if decode:
            (sa, sb, hh, ra, rb, wa, wb, wra, wrb, bb, dw, db, o) = refs
        else:
            (sa, sb, hh, ra, rb, wa, wb, wra, wrb, bb, o) = refs
        tslot = pl.program_id(0)
        a = sa[0] * ra[0]
        bmsg = sb[0] * rb[0]
        wr = wra[0, 0] + wrb[0, 0]
        bias = bb[0, tslot] + bb[0, 3 - tslot]
        halves = []
        for lo in (0, H):
            t = jnp.dot(a[:, lo:lo + H], wa[0, 0],
                        preferred_element_type=jnp.float32)
            t = t + jnp.dot(bmsg[:, lo:lo + H], wb[0, 0],
                            preferred_element_type=jnp.float32)
            t = t + jnp.dot(hh[0][:, lo:lo + H], wr,
                            preferred_element_type=jnp.float32)
            t = jnp.maximum(t + bias, 0.0)
            if decode:
                t = jnp.dot(t, dw[0], preferred_element_type=jnp.float32) \
                    + db[0]
            halves.append(t)
        o[0] = jnp.concatenate(halves, axis=1)

    in_specs = [
        pl.BlockSpec((1, _BP, 2 * H), lambda t, i: (t, i, 0)),
        pl.BlockSpec((1, _BP, 2 * H), lambda t, i: (t, i, 0)),
        pl.BlockSpec((1, _BP, 2 * H), lambda t, i: (t, i, 0)),
        pl.BlockSpec((1, _BP, 2 * H), lambda t, i: (t, i, 0)),
        pl.BlockSpec((1, _BP, 2 * H), lambda t, i: (t, i, 0)),
        pl.BlockSpec((1, 1, H, H), lambda t, i: (l, t, 0, 0)),
        pl.BlockSpec((1, 1, H, H), lambda t, i: (l, 3 - t, 0, 0)),
        pl.BlockSpec((1, 1, H, H), lambda t, i: (l, t, 0, 0)),
        pl.BlockSpec((1, 1, H, H), lambda t, i: (l, 3 - t, 0, 0)),
        pl.BlockSpec((1, 4, H), lambda t, i: (l, 0, 0)),
    ]
    args = [sA, sB, h, raA, raB, Wl, Wl, Wr, Wr, bl]
    if decode:
        in_specs += [
            pl.BlockSpec((1, H, OUT_DIM), lambda t, i: (t, 0, 0)),
            pl.BlockSpec((1, 1, OUT_DIM), lambda t, i: (t, 0, 0)),
        ]
        args += [decW, decb]
        out_spec = pl.BlockSpec((1, _BP, 2 * OUT_DIM), lambda t, i: (t, i, 0))
        out_shape = jax.ShapeDtypeStruct((2, N_NODES // 2, 2 * OUT_DIM),
                                         jnp.float32)
    else:
        out_spec = pl.BlockSpec((1, _BP, 2 * H), lambda t, i: (t, i, 0))
        out_shape = jax.ShapeDtypeStruct((2, _NPR, 2 * H), jnp.float32)

    return pl.pallas_call(
        body,
        grid=(2, N_NODES // _BN),
        in_specs=in_specs,
        out_specs=out_spec,
        out_shape=out_shape,
    )(*args)


def _ra_paired(cnt):
    ra = 1.0 / jnp.maximum(cnt[..., 0:1], 1.0)
    ra = ra.reshape(2, _NPR, 2, 1)
    return jnp.broadcast_to(ra, (2, _NPR, 2, H)).reshape(2, _NPR, 2 * H)


def _pad_edges(ei):
    npad = _EROWS * _C - E
    tail = N_NODES + (jnp.arange(npad, dtype=jnp.int32)
                      % (_NACC - N_NODES))
    tail = jnp.broadcast_to(tail, (2, npad))
    return jnp.concatenate([ei.astype(jnp.int32), tail],
                           axis=1).reshape(2, _EROWS, _C)


def kernel(x_primal, x_dual, edge_index_pp, edge_index_dd, edge_index_pd,
           enc_p_W, enc_p_b, enc_d_W, enc_d_b, Wl, bl, Wr,
           dec_p_W, dec_p_b, dec_d_W, dec_d_b):
    epp = _pad_edges(edge_index_pp)
    edd = _pad_edges(edge_index_dd)
    epd = _pad_edges(edge_index_pd)

    zeros64 = jnp.zeros((_NACC, H), jnp.float32)
    zeros8 = jnp.zeros((_NACC, _CW), jnp.float32)
    ones8 = jnp.ones((_C, _CW), jnp.float32)

    x_st = jnp.stack([x_primal, x_dual]).reshape(2, N_NODES // 2, 2 * IN_DIM)
    encW = jnp.stack([enc_p_W, enc_d_W])
    encb = jnp.stack([enc_p_b, enc_d_b]).reshape(2, 1, H)
    h = _enc(x_st, encW, encb)

    decW = jnp.stack([dec_p_W, dec_d_W])
    decb = jnp.stack([dec_p_b, dec_d_b]).reshape(2, 1, OUT_DIM)

    def unpair(a):
        return a.reshape(2, _NACC, H)

    def pair(a):
        return a.reshape(2, _NPR, 2 * H)

    sA, sB, cA, cB = _sc_segsum0(unpair(h), epp, edd, epd,
                                 zeros64, zeros8, ones8)
    raA, raB = _ra_paired(cA), _ra_paired(cB)
    h = _combine(0, pair(sA), pair(sB), h, raA, raB, Wl, bl, Wr)
    sA, sB = _sc_segsum1(unpair(h), epp, edd, epd, zeros64)
    out = _combine(1, pair(sA), pair(sB), h, raA, raB, Wl, bl, Wr,
                   decW=decW, decb=decb)
    out = out.reshape(2, N_NODES, OUT_DIM)
    return (out[0], out[1])
